# Initial kernel scaffold; baseline (speedup 1.0000x reference)
import jax
import jax.numpy as jnp
from jax.experimental import pallas as pl


def kernel(x):
    raise NotImplementedError("write your pallas kernel here")


# speedup vs baseline: 658.7879x; 658.7879x over previous
import jax
import jax.numpy as jnp
from jax import lax
from jax.experimental import pallas as pl
from jax.experimental.pallas import tpu as pltpu

_GW0, _GW1, _GW2 = 0.2989, 0.587, 0.114
_LOW_T, _HIGH_T = 20.0, 50.0
_TAN225 = 0.41421356237309503

_H = _W = 1024
_BR = 128
_NBLK = _H // _BR


def _front_kernel(xp_ref, strong_ref, weak_ref, hm_ref):
    i = pl.program_id(0)
    r0 = i * _BR
    xs = xp_ref[:, pl.ds(r0, _BR + 4), :]

    w0 = jnp.asarray(_GW0, jnp.bfloat16).astype(jnp.float32)
    w1 = jnp.asarray(_GW1, jnp.bfloat16).astype(jnp.float32)
    w2 = jnp.asarray(_GW2, jnp.bfloat16).astype(jnp.float32)
    xb = xs.astype(jnp.bfloat16).astype(jnp.float32)
    p0, p1, p2 = xb[0] * w0, xb[1] * w1, xb[2] * w2

    def two_sum(a, b):
        s = a + b
        bp = s - a
        ap = s - bp
        return s, (a - ap) + (b - bp)

    s1, e1 = two_sum(p0, p1)
    s2, e2 = two_sum(s1, p2)
    t, terr = two_sum(e1, e2)
    ti = pltpu.bitcast(t, jnp.uint32)
    away = jnp.where((t > 0) == (terr > 0), jnp.uint32(1),
                     jnp.uint32(0xFFFFFFFF))
    todd = jnp.where((terr == 0) | ((ti & 1) == 1), t,
                     pltpu.bitcast(ti + away, jnp.float32))
    gray = s2 + todd

    gs = gray.astype(jnp.bfloat16).astype(jnp.float32)

    a00 = gs[0:_BR + 2, 0:_W + 2]
    a01 = gs[0:_BR + 2, 1:_W + 3]
    a02 = gs[0:_BR + 2, 2:_W + 4]
    a10 = gs[1:_BR + 3, 0:_W + 2]
    a12 = gs[1:_BR + 3, 2:_W + 4]
    a20 = gs[2:_BR + 4, 0:_W + 2]
    a21 = gs[2:_BR + 4, 1:_W + 3]
    a22 = gs[2:_BR + 4, 2:_W + 4]
    gx = (a02 - a00) + 2.0 * (a12 - a10) + (a22 - a20)
    gy = (a00 - a20) + 2.0 * (a01 - a21) + (a02 - a22)
    mag = jnp.sqrt(gx * gx + gy * gy)

    c = mag[1:_BR + 1, 1:_W + 1]
    nU = mag[0:_BR, 1:_W + 1]
    nD = mag[2:_BR + 2, 1:_W + 1]
    nL = mag[1:_BR + 1, 0:_W]
    nR = mag[1:_BR + 1, 2:_W + 2]
    nUL = mag[0:_BR, 0:_W]
    nUR = mag[0:_BR, 2:_W + 2]
    nDL = mag[2:_BR + 2, 0:_W]
    nDR = mag[2:_BR + 2, 2:_W + 2]
    fx = gx[1:_BR + 1, 1:_W + 1]
    fy = gy[1:_BR + 1, 1:_W + 1]

    def ind(cond):
        return jnp.where(cond, 1.0, 0.0)

    txp = _TAN225 * fx
    typ = _TAN225 * fy
    b1 = jnp.maximum(ind(fx > 0) * ind(-txp <= fy) * ind(fy < txp),
                     ind(fx == 0) * ind(fy == 0))
    b2 = ind(fx > typ) * ind(fy >= txp)
    b3 = ind(fy > 0) * ind(-typ < fx) * ind(fx <= typ)
    b4 = ind(fx <= -typ) * ind(fy > -txp)
    b0 = 1.0 - (b1 + b2 + b3 + b4)

    keep_h = ind(c >= nL) * ind(c >= nR)
    keep_v = ind(c >= nU) * ind(c >= nD)
    keep_d1 = ind(c >= nUL) * ind(c >= nDR)
    keep_d2 = ind(c >= nUR) * ind(c >= nDL)
    keep = ((b1 + b3) * keep_v + b2 * keep_d1
            + b4 * keep_d2 + b0 * keep_h)

    grow = r0 + lax.broadcasted_iota(jnp.int32, (_BR, _W), 0)
    gcol = lax.broadcasted_iota(jnp.int32, (_BR, _W), 1)
    interior = (ind(grow >= 1) * ind(grow <= _H - 2)
                * ind(gcol >= 1) * ind(gcol <= _W - 2))
    supp = c * keep * interior

    strong = jnp.where(supp >= _HIGH_T, 1.0, 0.0)
    weak = jnp.where(supp >= _LOW_T, 1.0, 0.0) - strong
    strong_ref[...] = strong
    weak_ref[...] = weak

    zc = jnp.zeros((_BR, 1), jnp.float32)
    sl = jnp.concatenate([strong[:, 1:], zc], axis=1)
    sr = jnp.concatenate([zc, strong[:, :-1]], axis=1)
    hm_ref[...] = jnp.maximum(strong, jnp.maximum(sl, sr))


def _hyst_kernel(strong_ref, weak_ref, hm_ref, out_ref):
    out_ref[...] = strong_ref[...]
    lane = lax.broadcasted_iota(jnp.int32, (1, _W), 1)
    emask = (jnp.where(lane >= 1, 1.0, 0.0)
             * jnp.where(lane <= _W - 2, 1.0, 0.0))

    def body(i, carry):
        prev = out_ref[pl.ds(i - 1, 1)][0]
        zc = jnp.zeros((1, 1), jnp.float32)
        p_l = jnp.concatenate([prev[:, 1:], zc], axis=1)
        p_r = jnp.concatenate([zc, prev[:, :-1]], axis=1)
        hmp = jnp.maximum(prev, jnp.maximum(p_l, p_r))
        base = jnp.maximum(hmp, jnp.maximum(hm_ref[pl.ds(i, 1)][0],
                                            hm_ref[pl.ds(i + 1, 1)][0]))
        wk = weak_ref[pl.ds(i, 1)][0]
        b = wk * base * emask
        m = wk * emask
        for s in (1, 2, 4, 8, 16, 32, 64, 128, 256, 512):
            bs = jnp.concatenate(
                [jnp.zeros((1, s), jnp.float32), b[:, :_W - s]], axis=1)
            ms = jnp.concatenate(
                [jnp.ones((1, s), jnp.float32), m[:, :_W - s]], axis=1)
            b = jnp.maximum(b, m * bs)
            m = m * ms
        new_row = jnp.maximum(strong_ref[pl.ds(i, 1)][0], b)
        out_ref[pl.ds(i, 1)] = new_row[None]
        return carry

    lax.fori_loop(1, _H - 1, body, 0)


def _front(xp):
    return pl.pallas_call(
        _front_kernel,
        grid=(_NBLK,),
        in_specs=[pl.BlockSpec((3, _H + 4, _W + 4), lambda i: (0, 0, 0))],
        out_specs=[pl.BlockSpec((_BR, _W), lambda i: (i, 0))] * 3,
        out_shape=[jax.ShapeDtypeStruct((_H, _W), jnp.float32)] * 3,
        compiler_params=pltpu.CompilerParams(
            dimension_semantics=("parallel",),
        ),
        name="canny_front",
    )(xp)


def _hyst(s3, w3, h3):
    return pl.pallas_call(
        _hyst_kernel,
        out_shape=jax.ShapeDtypeStruct((_H, 1, _W), jnp.float32),
        name="canny_hyst",
    )(s3, w3, h3)


def kernel(x):
    xp = jnp.pad(x[0], ((0, 0), (2, 2), (2, 2)))
    strong, weak, hm = _front(xp)
    edges = _hyst(strong.reshape(_H, 1, _W),
                  weak.reshape(_H, 1, _W),
                  hm.reshape(_H, 1, _W))
    return edges.reshape(1, 1, _H, _W)
